# baseline (speedup 1.0000x reference)
import functools

import jax
import jax.numpy as jnp
from jax import lax
from jax.experimental import pallas as pl
from jax.experimental.pallas import tpu as pltpu
from jax.experimental.pallas import tpu_sc as plsc

N = 10000
E = 320000
D_IN = 128
D_HID = 256
D_OUT = 64

NC = 2
NS = 16
NW = NC * NS
EW = E // NW
K = 50
NCH = EW // K
SB = 40
NST = NCH // SB
NBUF = 4
NP = 10240
RS = NP // NS

_MESH = plsc.VectorSubcoreMesh(
    core_axis_name="c", subcore_axis_name="s", num_cores=NC, num_subcores=NS
)


def _fill(buf, rows, d, value):
    v = jnp.full((16,), value, jnp.float32)

    @pl.loop(0, rows)
    def _(r):
        @pl.loop(0, d, step=16)
        def _(c):
            buf[r, pl.ds(c, 16)] = v


def _init_acc(zsrc, acc_sh, sid):
    @pl.loop(0, RS, step=80)
    def _(r):
        pltpu.sync_copy(zsrc.at[pl.ds(0, 80)], acc_sh.at[pl.ds(sid * RS + r, 80)])


@functools.partial(
    pl.kernel,
    out_type=jax.ShapeDtypeStruct((NC, NP, 16), jnp.float32),
    mesh=_MESH,
    scratch_types=[
        pltpu.VMEM((NCH, K), jnp.int32),
        pltpu.VMEM((K, 16), jnp.float32),
        pltpu.VMEM_SHARED((NP, 16), jnp.float32),
        pltpu.SemaphoreType.DMA,
    ],
    name="gcn_degree_sc",
)
def _deg_kernel(edge_hbm, out_hbm, didx_v, ones_v, acc_sh, sem):
    cid = lax.axis_index("c")
    sid = lax.axis_index("s")
    w = cid * NS + sid
    pltpu.sync_copy(edge_hbm.at[1, pl.ds(w * NCH, NCH)], didx_v)

    _fill(ones_v, K, 16, 0.0)
    _init_acc(ones_v, acc_sh, sid)
    _fill(ones_v, K, 16, 1.0)
    plsc.subcore_barrier()

    @pl.loop(0, NCH)
    def _(i):
        pltpu.async_copy(ones_v, acc_sh.at[didx_v.at[i]], sem, add=True)

    @pl.loop(0, NCH)
    def _(i):
        pltpu.make_async_copy(ones_v, acc_sh.at[didx_v.at[i]], sem).wait()

    plsc.subcore_barrier()
    pltpu.sync_copy(
        acc_sh.at[pl.ds(sid * RS, RS)], out_hbm.at[cid].at[pl.ds(sid * RS, RS)]
    )


def _make_prop(d, label):
    @functools.partial(
        pl.kernel,
        out_type=jax.ShapeDtypeStruct((NC, NP, d), jnp.float32),
        mesh=_MESH,
        scratch_types=[
            pltpu.VMEM((SB, K), jnp.int32),
            pltpu.VMEM((SB, K), jnp.int32),
            [pltpu.VMEM((K, d), jnp.float32) for _ in range(NBUF)],
            pltpu.VMEM_SHARED((NP, d), jnp.float32),
            [pltpu.SemaphoreType.DMA for _ in range(NBUF)],
        ],
        name=label,
    )
    def prop(tab_hbm, edge_hbm, out_hbm,
             sidx_v, didx_v, rows, acc_sh, sems):
        cid = lax.axis_index("c")
        sid = lax.axis_index("s")
        w = cid * NS + sid

        _fill(rows[0], K, d, 0.0)
        _init_acc(rows[0], acc_sh, sid)
        plsc.subcore_barrier()

        def start_gather(i, b):
            pltpu.async_copy(tab_hbm.at[sidx_v.at[i]], rows[b], sems[b])

        def wait_gather(i, b):
            pltpu.make_async_copy(tab_hbm.at[sidx_v.at[i]], rows[b], sems[b]).wait()

        def start_scatter(i, b):
            pltpu.async_copy(rows[b], acc_sh.at[didx_v.at[i]], sems[b], add=True)

        def wait_scatter(i, b):
            pltpu.make_async_copy(rows[b], acc_sh.at[didx_v.at[i]], sems[b]).wait()

        for st in range(NST):
            base = w * NCH + st * SB
            pltpu.sync_copy(edge_hbm.at[0, pl.ds(base, SB)], sidx_v)
            pltpu.sync_copy(edge_hbm.at[1, pl.ds(base, SB)], didx_v)

            for b in range(NBUF):
                start_gather(b, b)

            @pl.loop(0, SB // NBUF - 1)
            def _(j):
                i = NBUF * j
                for b in range(NBUF):
                    wait_gather(i + b, b)
                    start_scatter(i + b, b)
                for b in range(NBUF):
                    wait_scatter(i + b, b)
                    start_gather(i + b + NBUF, b)

            i0 = SB - NBUF
            for b in range(NBUF):
                wait_gather(i0 + b, b)
                start_scatter(i0 + b, b)
            for b in range(NBUF):
                wait_scatter(i0 + b, b)

        plsc.subcore_barrier()
        pltpu.sync_copy(
            acc_sh.at[pl.ds(sid * RS, RS)],
            out_hbm.at[cid].at[pl.ds(sid * RS, RS)],
        )

    return prop


_prop128 = _make_prop(D_IN, "gcn_prop128_sc")

_R = 1000
_TILES = N // _R


def _dinv_of(degp_ref):
    deg = degp_ref[0, :, 0:1] + degp_ref[1, :, 0:1] + 1.0
    return lax.rsqrt(deg)


def _scale_body(degp_ref, x_ref, xs_ref):
    xs_ref[...] = x_ref[...] * _dinv_of(degp_ref)


def _mid_body(degp_ref, aggp_ref, xs_ref, w1_ref, b1_ref, w2_ref, gs_ref):
    dinv = _dinv_of(degp_ref)
    a = (aggp_ref[0] + aggp_ref[1] + xs_ref[...]) * dinv
    h = jnp.dot(a, w1_ref[...], preferred_element_type=jnp.float32) + b1_ref[...]
    h = jnp.maximum(h, 0.0)
    g = jnp.dot(h, w2_ref[...], preferred_element_type=jnp.float32)
    gs_ref[...] = jnp.concatenate(
        [g * dinv, jnp.zeros_like(g)], axis=1)


def _final_body(degp_ref, aggp_ref, gs_ref, b2_ref, out_ref):
    dinv = _dinv_of(degp_ref)
    agg = aggp_ref[0, :, 0:D_OUT] + aggp_ref[1, :, 0:D_OUT] + gs_ref[:, 0:D_OUT]
    out_ref[...] = agg * dinv + b2_ref[...]


def _deg_spec():
    return pl.BlockSpec((NC, _R, 16), lambda i: (0, i, 0))


def _rows_spec(d):
    return pl.BlockSpec((_R, d), lambda i: (i, 0))


def _part_spec(d):
    return pl.BlockSpec((NC, _R, d), lambda i: (0, i, 0))


def _full_spec(shape):
    return pl.BlockSpec(shape, lambda i: tuple(0 for _ in shape))


def kernel(x, edge_index, W1, b1, W2, b2):
    edges = edge_index.reshape(2, E // K, K)

    degp = _deg_kernel(edges)

    xs = pl.pallas_call(
        _scale_body,
        grid=(_TILES,),
        in_specs=[_deg_spec(), _rows_spec(D_IN)],
        out_specs=_rows_spec(D_IN),
        out_shape=jax.ShapeDtypeStruct((N, D_IN), jnp.float32),
    )(degp, x)

    aggp1 = _prop128(xs, edges)

    gs = pl.pallas_call(
        _mid_body,
        grid=(_TILES,),
        in_specs=[
            _deg_spec(),
            _part_spec(D_IN),
            _rows_spec(D_IN),
            _full_spec((D_IN, D_HID)),
            _full_spec((1, D_HID)),
            _full_spec((D_HID, D_OUT)),
        ],
        out_specs=_rows_spec(2 * D_OUT),
        out_shape=jax.ShapeDtypeStruct((N, 2 * D_OUT), jnp.float32),
    )(degp, aggp1, xs, W1, b1.reshape(1, D_HID), W2)

    aggp2 = _prop128(gs, edges)

    out = pl.pallas_call(
        _final_body,
        grid=(_TILES,),
        in_specs=[
            _deg_spec(),
            _part_spec(2 * D_OUT),
            _rows_spec(2 * D_OUT),
            _full_spec((1, D_OUT)),
        ],
        out_specs=_rows_spec(D_OUT),
        out_shape=jax.ShapeDtypeStruct((N, D_OUT), jnp.float32),
    )(degp, aggp2, gs, b2.reshape(1, D_OUT))

    return out


# speedup vs baseline: 1.1589x; 1.1589x over previous
import functools

import jax
import jax.numpy as jnp
from jax import lax
from jax.experimental import pallas as pl
from jax.experimental.pallas import tpu as pltpu
from jax.experimental.pallas import tpu_sc as plsc

N = 10000
E = 320000
D_IN = 128
D_HID = 256
D_OUT = 64

NC = 2
NS = 16
NW = NC * NS
EW = E // NW
K = 50
NCH = EW // K
SB = 40
NST = NCH // SB
NBUF = 4
NP = 10240
RS = NP // NS

_MESH = plsc.VectorSubcoreMesh(
    core_axis_name="c", subcore_axis_name="s", num_cores=NC, num_subcores=NS
)


def _fill(buf, rows, d, value):
    v = jnp.full((16,), value, jnp.float32)

    @pl.loop(0, rows)
    def _(r):
        @pl.loop(0, d, step=16)
        def _(c):
            buf[r, pl.ds(c, 16)] = v


def _init_acc(zsrc, acc_sh, sid):
    @pl.loop(0, RS, step=80)
    def _(r):
        pltpu.sync_copy(zsrc.at[pl.ds(0, 80)], acc_sh.at[pl.ds(sid * RS + r, 80)])


@functools.partial(
    pl.kernel,
    out_type=jax.ShapeDtypeStruct((NC, NP, 16), jnp.float32),
    mesh=_MESH,
    scratch_types=[
        pltpu.VMEM((NCH, K), jnp.int32),
        pltpu.VMEM((K, 16), jnp.float32),
        pltpu.VMEM_SHARED((NP, 16), jnp.float32),
        pltpu.SemaphoreType.DMA,
    ],
    name="gcn_degree_sc",
)
def _deg_kernel(edge_hbm, out_hbm, didx_v, ones_v, acc_sh, sem):
    cid = lax.axis_index("c")
    sid = lax.axis_index("s")
    w = cid * NS + sid
    pltpu.sync_copy(edge_hbm.at[1, pl.ds(w * NCH, NCH)], didx_v)

    _fill(ones_v, K, 16, 0.0)
    _init_acc(ones_v, acc_sh, sid)
    _fill(ones_v, K, 16, 1.0)
    plsc.subcore_barrier()

    @pl.loop(0, NCH)
    def _(i):
        pltpu.async_copy(ones_v, acc_sh.at[didx_v.at[i]], sem, add=True)

    @pl.loop(0, NCH)
    def _(i):
        pltpu.make_async_copy(ones_v, acc_sh.at[didx_v.at[i]], sem).wait()

    plsc.subcore_barrier()
    pltpu.sync_copy(
        acc_sh.at[pl.ds(sid * RS, RS)], out_hbm.at[cid].at[pl.ds(sid * RS, RS)]
    )


def _make_prop(d, label):
    @functools.partial(
        pl.kernel,
        out_type=jax.ShapeDtypeStruct((NC, NP, d), jnp.float32),
        mesh=_MESH,
        scratch_types=[
            pltpu.VMEM((2, SB, K), jnp.int32),
            pltpu.VMEM((2, SB, K), jnp.int32),
            [pltpu.VMEM((K, d), jnp.float32) for _ in range(NBUF)],
            pltpu.VMEM_SHARED((NP, d), jnp.float32),
            [pltpu.SemaphoreType.DMA for _ in range(NBUF)],
            pltpu.SemaphoreType.DMA,
        ],
        name=label,
    )
    def prop(tab_hbm, edge_hbm, out_hbm,
             sidx_v, didx_v, rows, acc_sh, sems, psem):
        cid = lax.axis_index("c")
        sid = lax.axis_index("s")
        w = cid * NS + sid

        _fill(rows[0], K, d, 0.0)
        _init_acc(rows[0], acc_sh, sid)
        plsc.subcore_barrier()

        def start_gather(p, i, b):
            pltpu.async_copy(tab_hbm.at[sidx_v.at[p].at[i]], rows[b], sems[b])

        def wait_gather(p, i, b):
            pltpu.make_async_copy(
                tab_hbm.at[sidx_v.at[p].at[i]], rows[b], sems[b]).wait()

        def scatter_add(p, i, b):
            pltpu.sync_copy(rows[b], acc_sh.at[didx_v.at[p].at[i]], add=True)

        def idx_copies(st, p):
            base = w * NCH + st * SB
            return (
                pltpu.make_async_copy(
                    edge_hbm.at[0, pl.ds(base, SB)], sidx_v.at[p], psem),
                pltpu.make_async_copy(
                    edge_hbm.at[1, pl.ds(base, SB)], didx_v.at[p], psem),
            )

        for cp in idx_copies(0, 0):
            cp.start()
            cp.wait()

        for st in range(NST):
            p = st % 2
            if st + 1 < NST:
                pre = idx_copies(st + 1, 1 - p)
                for cp in pre:
                    cp.start()

            for b in range(NBUF):
                start_gather(p, b, b)

            @pl.loop(0, SB // NBUF - 1)
            def _(j):
                i = NBUF * j
                for b in range(NBUF):
                    wait_gather(p, i + b, b)
                    scatter_add(p, i + b, b)
                    start_gather(p, i + b + NBUF, b)

            i0 = SB - NBUF
            for b in range(NBUF):
                wait_gather(p, i0 + b, b)
                scatter_add(p, i0 + b, b)

            if st + 1 < NST:
                for cp in pre:
                    cp.wait()

        plsc.subcore_barrier()
        pltpu.sync_copy(
            acc_sh.at[pl.ds(sid * RS, RS)],
            out_hbm.at[cid].at[pl.ds(sid * RS, RS)],
        )

    return prop


_prop128 = _make_prop(D_IN, "gcn_prop128_sc")

_R = 1000
_TILES = N // _R


def _dinv_of(degp_ref):
    deg = degp_ref[0, :, 0:1] + degp_ref[1, :, 0:1] + 1.0
    return lax.rsqrt(deg)


def _scale_body(degp_ref, x_ref, xs_ref):
    xs_ref[...] = x_ref[...] * _dinv_of(degp_ref)


def _mid_body(degp_ref, aggp_ref, xs_ref, w1_ref, b1_ref, w2_ref, gs_ref):
    dinv = _dinv_of(degp_ref)
    a = (aggp_ref[0] + aggp_ref[1] + xs_ref[...]) * dinv
    h = jnp.dot(a, w1_ref[...], preferred_element_type=jnp.float32) + b1_ref[...]
    h = jnp.maximum(h, 0.0)
    g = jnp.dot(h, w2_ref[...], preferred_element_type=jnp.float32)
    gs_ref[...] = jnp.concatenate(
        [g * dinv, jnp.zeros_like(g)], axis=1)


def _final_body(degp_ref, aggp_ref, gs_ref, b2_ref, out_ref):
    dinv = _dinv_of(degp_ref)
    agg = aggp_ref[0, :, 0:D_OUT] + aggp_ref[1, :, 0:D_OUT] + gs_ref[:, 0:D_OUT]
    out_ref[...] = agg * dinv + b2_ref[...]


def _deg_spec():
    return pl.BlockSpec((NC, _R, 16), lambda i: (0, i, 0))


def _rows_spec(d):
    return pl.BlockSpec((_R, d), lambda i: (i, 0))


def _part_spec(d):
    return pl.BlockSpec((NC, _R, d), lambda i: (0, i, 0))


def _full_spec(shape):
    return pl.BlockSpec(shape, lambda i: tuple(0 for _ in shape))


def kernel(x, edge_index, W1, b1, W2, b2):
    edges = edge_index.reshape(2, E // K, K)

    degp = _deg_kernel(edges)

    xs = pl.pallas_call(
        _scale_body,
        grid=(_TILES,),
        in_specs=[_deg_spec(), _rows_spec(D_IN)],
        out_specs=_rows_spec(D_IN),
        out_shape=jax.ShapeDtypeStruct((N, D_IN), jnp.float32),
    )(degp, x)

    aggp1 = _prop128(xs, edges)

    gs = pl.pallas_call(
        _mid_body,
        grid=(_TILES,),
        in_specs=[
            _deg_spec(),
            _part_spec(D_IN),
            _rows_spec(D_IN),
            _full_spec((D_IN, D_HID)),
            _full_spec((1, D_HID)),
            _full_spec((D_HID, D_OUT)),
        ],
        out_specs=_rows_spec(2 * D_OUT),
        out_shape=jax.ShapeDtypeStruct((N, 2 * D_OUT), jnp.float32),
    )(degp, aggp1, xs, W1, b1.reshape(1, D_HID), W2)

    aggp2 = _prop128(gs, edges)

    out = pl.pallas_call(
        _final_body,
        grid=(_TILES,),
        in_specs=[
            _deg_spec(),
            _part_spec(2 * D_OUT),
            _rows_spec(2 * D_OUT),
            _full_spec((1, D_OUT)),
        ],
        out_specs=_rows_spec(D_OUT),
        out_shape=jax.ShapeDtypeStruct((N, D_OUT), jnp.float32),
    )(degp, aggp2, gs, b2.reshape(1, D_OUT))

    return out


# speedup vs baseline: 1.1996x; 1.0351x over previous
import functools

import jax
import jax.numpy as jnp
from jax import lax
from jax.experimental import pallas as pl
from jax.experimental.pallas import tpu as pltpu
from jax.experimental.pallas import tpu_sc as plsc

N = 10000
E = 320000
D_IN = 128
D_HID = 256
D_OUT = 64

NC = 2
NS = 16
NW = NC * NS
EW = E // NW
K = 50
NCH = EW // K
SB = 40
NST = NCH // SB
NBUF = 4
NP = 10240
RS = NP // NS

_MESH = plsc.VectorSubcoreMesh(
    core_axis_name="c", subcore_axis_name="s", num_cores=NC, num_subcores=NS
)


def _fill(buf, rows, d, value):
    v = jnp.full((16,), value, jnp.float32)

    @pl.loop(0, rows)
    def _(r):
        @pl.loop(0, d, step=16)
        def _(c):
            buf[r, pl.ds(c, 16)] = v


def _init_acc(zsrc, acc_sh, sid):
    @pl.loop(0, RS, step=80)
    def _(r):
        pltpu.sync_copy(zsrc.at[pl.ds(0, 80)], acc_sh.at[pl.ds(sid * RS + r, 80)])


@functools.partial(
    pl.kernel,
    out_type=jax.ShapeDtypeStruct((NC, NP, 16), jnp.float32),
    mesh=_MESH,
    scratch_types=[
        pltpu.VMEM((NCH, K), jnp.int32),
        pltpu.VMEM((K, 16), jnp.float32),
        pltpu.VMEM_SHARED((NP, 16), jnp.float32),
        pltpu.SemaphoreType.DMA,
    ],
    name="gcn_degree_sc",
)
def _deg_kernel(edge_hbm, out_hbm, didx_v, ones_v, acc_sh, sem):
    cid = lax.axis_index("c")
    sid = lax.axis_index("s")
    w = cid * NS + sid
    pltpu.sync_copy(edge_hbm.at[1, pl.ds(w * NCH, NCH)], didx_v)

    _fill(ones_v, K, 16, 0.0)
    _init_acc(ones_v, acc_sh, sid)
    _fill(ones_v, K, 16, 1.0)
    plsc.subcore_barrier()

    @pl.loop(0, NCH)
    def _(i):
        pltpu.async_copy(ones_v, acc_sh.at[didx_v.at[i]], sem, add=True)

    @pl.loop(0, NCH)
    def _(i):
        pltpu.make_async_copy(ones_v, acc_sh.at[didx_v.at[i]], sem).wait()

    plsc.subcore_barrier()
    pltpu.sync_copy(
        acc_sh.at[pl.ds(sid * RS, RS)], out_hbm.at[cid].at[pl.ds(sid * RS, RS)]
    )


def _make_prop(d, label):
    @functools.partial(
        pl.kernel,
        out_type=jax.ShapeDtypeStruct((NC, NP, d), jnp.float32),
        mesh=_MESH,
        scratch_types=[
            pltpu.VMEM((2, SB, K), jnp.int32),
            pltpu.VMEM((2, SB, K), jnp.int32),
            [pltpu.VMEM((K, d), jnp.float32) for _ in range(NBUF)],
            pltpu.VMEM_SHARED((NP, d), jnp.float32),
            [pltpu.SemaphoreType.DMA for _ in range(NBUF)],
            pltpu.SemaphoreType.DMA,
        ],
        name=label,
    )
    def prop(tab_hbm, edge_hbm, out_hbm,
             sidx_v, didx_v, rows, acc_sh, sems, psem):
        cid = lax.axis_index("c")
        sid = lax.axis_index("s")
        w = cid * NS + sid

        _fill(rows[0], K, d, 0.0)
        _init_acc(rows[0], acc_sh, sid)
        plsc.subcore_barrier()

        def start_gather(p, i, b):
            pltpu.async_copy(tab_hbm.at[sidx_v.at[p].at[i]], rows[b], sems[b])

        def wait_gather(p, i, b):
            pltpu.make_async_copy(
                tab_hbm.at[sidx_v.at[p].at[i]], rows[b], sems[b]).wait()

        def scatter_add(p, i, b):
            pltpu.sync_copy(rows[b], acc_sh.at[didx_v.at[p].at[i]], add=True)

        def idx_copies(st, p):
            base = w * NCH + st * SB
            return (
                pltpu.make_async_copy(
                    edge_hbm.at[0, pl.ds(base, SB)], sidx_v.at[p], psem),
                pltpu.make_async_copy(
                    edge_hbm.at[1, pl.ds(base, SB)], didx_v.at[p], psem),
            )

        for cp in idx_copies(0, 0):
            cp.start()
            cp.wait()

        for b in range(NBUF):
            start_gather(0, b, b)

        for st in range(NST):
            p = st % 2
            if st + 1 < NST:
                pre = idx_copies(st + 1, 1 - p)
                for cp in pre:
                    cp.start()

            @pl.loop(0, SB // NBUF - 1)
            def _(j):
                i = NBUF * j
                for b in range(NBUF):
                    wait_gather(p, i + b, b)
                    scatter_add(p, i + b, b)
                    start_gather(p, i + b + NBUF, b)

            if st + 1 < NST:
                for cp in pre:
                    cp.wait()

            i0 = SB - NBUF
            for b in range(NBUF):
                wait_gather(p, i0 + b, b)
                scatter_add(p, i0 + b, b)
                if st + 1 < NST:
                    start_gather(1 - p, b, b)

        plsc.subcore_barrier()
        pltpu.sync_copy(
            acc_sh.at[pl.ds(sid * RS, RS)],
            out_hbm.at[cid].at[pl.ds(sid * RS, RS)],
        )

    return prop


_prop128 = _make_prop(D_IN, "gcn_prop128_sc")

_R = 1000
_TILES = N // _R


def _dinv_of(degp_ref):
    deg = degp_ref[0, :, 0:1] + degp_ref[1, :, 0:1] + 1.0
    return lax.rsqrt(deg)


def _scale_body(degp_ref, x_ref, xs_ref):
    xs_ref[...] = x_ref[...] * _dinv_of(degp_ref)


def _mid_body(degp_ref, aggp_ref, xs_ref, w1_ref, b1_ref, w2_ref, gs_ref):
    dinv = _dinv_of(degp_ref)
    a = (aggp_ref[0] + aggp_ref[1] + xs_ref[...]) * dinv
    h = jnp.dot(a, w1_ref[...], preferred_element_type=jnp.float32) + b1_ref[...]
    h = jnp.maximum(h, 0.0)
    g = jnp.dot(h, w2_ref[...], preferred_element_type=jnp.float32)
    gs_ref[...] = jnp.concatenate(
        [g * dinv, jnp.zeros_like(g)], axis=1)


def _final_body(degp_ref, aggp_ref, gs_ref, b2_ref, out_ref):
    dinv = _dinv_of(degp_ref)
    agg = aggp_ref[0, :, 0:D_OUT] + aggp_ref[1, :, 0:D_OUT] + gs_ref[:, 0:D_OUT]
    out_ref[...] = agg * dinv + b2_ref[...]


def _deg_spec():
    return pl.BlockSpec((NC, _R, 16), lambda i: (0, i, 0))


def _rows_spec(d):
    return pl.BlockSpec((_R, d), lambda i: (i, 0))


def _part_spec(d):
    return pl.BlockSpec((NC, _R, d), lambda i: (0, i, 0))


def _full_spec(shape):
    return pl.BlockSpec(shape, lambda i: tuple(0 for _ in shape))


def kernel(x, edge_index, W1, b1, W2, b2):
    edges = edge_index.reshape(2, E // K, K)

    degp = _deg_kernel(edges)

    xs = pl.pallas_call(
        _scale_body,
        grid=(_TILES,),
        in_specs=[_deg_spec(), _rows_spec(D_IN)],
        out_specs=_rows_spec(D_IN),
        out_shape=jax.ShapeDtypeStruct((N, D_IN), jnp.float32),
    )(degp, x)

    aggp1 = _prop128(xs, edges)

    gs = pl.pallas_call(
        _mid_body,
        grid=(_TILES,),
        in_specs=[
            _deg_spec(),
            _part_spec(D_IN),
            _rows_spec(D_IN),
            _full_spec((D_IN, D_HID)),
            _full_spec((1, D_HID)),
            _full_spec((D_HID, D_OUT)),
        ],
        out_specs=_rows_spec(2 * D_OUT),
        out_shape=jax.ShapeDtypeStruct((N, 2 * D_OUT), jnp.float32),
    )(degp, aggp1, xs, W1, b1.reshape(1, D_HID), W2)

    aggp2 = _prop128(gs, edges)

    out = pl.pallas_call(
        _final_body,
        grid=(_TILES,),
        in_specs=[
            _deg_spec(),
            _part_spec(2 * D_OUT),
            _rows_spec(2 * D_OUT),
            _full_spec((1, D_OUT)),
        ],
        out_specs=_rows_spec(D_OUT),
        out_shape=jax.ShapeDtypeStruct((N, D_OUT), jnp.float32),
    )(degp, aggp2, gs, b2.reshape(1, D_OUT))

    return out
